# baseline (speedup 1.0000x reference)
import functools

import jax
import jax.numpy as jnp
from jax import lax
from jax.experimental import pallas as pl
from jax.experimental.pallas import tpu as pltpu, tpu_sc as plsc

VOCAB = 1000000
DIM = 64
G_MIN = 0.1

N_ROWS = 4096 * 200

DT_K = 384
DT_PITCH = DT_K + 1
DT_CHUNKS = 2604
V_TAIL = DT_CHUNKS * DT_K
TAIL_N = (VOCAB - V_TAIL) * DIM

C = 256
SUB = 128
NSUB = C // SUB


def _mesh():
    return plsc.VectorSubcoreMesh(core_axis_name="c", subcore_axis_name="s")


def _build_detile():
    info = plsc.get_sparse_core_info()
    nc, ns = info.num_cores, info.num_subcores
    nw = nc * ns
    n_units = 2 * DT_CHUNKS

    @functools.partial(
        pl.kernel,
        mesh=_mesh(),
        compiler_params=pltpu.CompilerParams(needs_layout_passes=False),
        out_type=[
            jax.ShapeDtypeStruct((VOCAB * DIM,), jnp.float32),
            jax.ShapeDtypeStruct((VOCAB * DIM,), jnp.float32),
        ],
        scratch_types=[
            pltpu.VMEM((DIM, DT_PITCH), jnp.float32),
            pltpu.VMEM((DIM, DT_PITCH), jnp.float32),
            pltpu.VMEM((DT_K * DIM,), jnp.float32),
            pltpu.VMEM((DT_K * DIM,), jnp.float32),
            pltpu.SemaphoreType.DMA,
            pltpu.SemaphoreType.DMA,
            pltpu.SemaphoreType.DMA,
            pltpu.SemaphoreType.DMA,
        ],
    )
    def detile(baseT_h, priorT_h, btail_h, ptail_h, base_rm_h, prior_rm_h,
               in0, in1, ob0, ob1, si0, si1, so0, so1):
        wid = lax.axis_index("s") * nc + lax.axis_index("c")
        slots = ((in0, ob0, si0, so0), (in1, ob1, si1, so1))
        srcs = (baseT_h, priorT_h)
        dsts = (base_rm_h, prior_rm_h)

        @pl.when(wid == 0)
        def _():
            pltpu.async_copy(
                btail_h, base_rm_h.at[pl.ds(V_TAIL * DIM, TAIL_N)], si0)
            pltpu.async_copy(
                ptail_h, prior_rm_h.at[pl.ds(V_TAIL * DIM, TAIL_N)], so0)
            pltpu.make_async_copy(
                btail_h, base_rm_h.at[pl.ds(V_TAIL * DIM, TAIL_N)], si0).wait()
            pltpu.make_async_copy(
                ptail_h, prior_rm_h.at[pl.ds(V_TAIL * DIM, TAIL_N)], so0).wait()

        def fire_in(g, slot):
            inb, _, sem_i, _ = slot
            tb = g % 2
            v0 = (g // 2) * DT_K
            for tix in range(2):
                @pl.when(tb == tix)
                def _():
                    pltpu.async_copy(
                        srcs[tix].at[:, pl.ds(v0, DT_K)],
                        inb.at[:, pl.ds(0, DT_K)], sem_i)

        def wait_in(g, slot):
            inb, _, sem_i, _ = slot
            tb = g % 2
            v0 = (g // 2) * DT_K
            for tix in range(2):
                @pl.when(tb == tix)
                def _():
                    pltpu.make_async_copy(
                        srcs[tix].at[:, pl.ds(v0, DT_K)],
                        inb.at[:, pl.ds(0, DT_K)], sem_i).wait()

        lane = jnp.arange(16, dtype=jnp.int32)

        def transpose(slot):
            inb, ob, _, _ = slot

            def row(v, carry):
                cols = jnp.full((16,), 0, jnp.int32) + v
                for j in range(DIM // 16):
                    vals = plsc.load_gather(inb, [lane + j * 16, cols])
                    ob[pl.ds(v * DIM + j * 16, 16)] = vals
                return carry

            lax.fori_loop(0, DT_K, row, 0)

        def fire_out(g, slot):
            _, ob, _, sem_o = slot
            tb = g % 2
            v0 = (g // 2) * DT_K
            for tix in range(2):
                @pl.when(tb == tix)
                def _():
                    pltpu.async_copy(
                        ob, dsts[tix].at[pl.ds(v0 * DIM, DT_K * DIM)], sem_o)

        def wait_out(g, slot):
            _, ob, _, sem_o = slot
            tb = g % 2
            v0 = (g // 2) * DT_K
            for tix in range(2):
                @pl.when(tb == tix)
                def _():
                    pltpu.make_async_copy(
                        ob, dsts[tix].at[pl.ds(v0 * DIM, DT_K * DIM)],
                        sem_o).wait()

        n_u = (n_units - wid + nw - 1) // nw
        g_of = lambda u: wid + u * nw

        fire_in(g_of(0), slots[0])

        @pl.when(n_u > 1)
        def _():
            fire_in(g_of(1), slots[1])

        def pairstep(p, carry):
            u0 = 2 * p
            wait_in(g_of(u0), slots[0])
            transpose(slots[0])
            fire_out(g_of(u0), slots[0])

            @pl.when(u0 + 2 < n_u)
            def _():
                wait_out(g_of(u0), slots[0])
                fire_in(g_of(u0 + 2), slots[0])

            @pl.when(u0 + 1 < n_u)
            def _():
                wait_in(g_of(u0 + 1), slots[1])
                transpose(slots[1])
                fire_out(g_of(u0 + 1), slots[1])

                @pl.when(u0 + 3 < n_u)
                def _():
                    wait_out(g_of(u0 + 1), slots[1])
                    fire_in(g_of(u0 + 3), slots[1])

            return carry

        lax.fori_loop(0, (n_u + 1) // 2, pairstep, 0)

        @pl.when(n_u % 2 == 1)
        def _():
            wait_out(g_of(n_u - 1), slots[0])
            wait_out(g_of(n_u - 2), slots[1])

        @pl.when(n_u % 2 == 0)
        def _():
            wait_out(g_of(n_u - 1), slots[1])
            wait_out(g_of(n_u - 2), slots[0])

    return detile


def _build_gather():
    info = plsc.get_sparse_core_info()
    nc, ns = info.num_cores, info.num_subcores
    nw = nc * ns
    rows_per_w = N_ROWS // nw
    nchunks = rows_per_w // C
    npairs = nchunks // 2

    _DNUMS = lax.GatherDimensionNumbers(
        offset_dims=(), collapsed_slice_dims=(0,), start_index_map=(0,))

    @functools.partial(
        pl.kernel,
        mesh=_mesh(),
        compiler_params=pltpu.CompilerParams(use_tc_tiling_on_sc=False),
        out_type=jax.ShapeDtypeStruct((N_ROWS, DIM), jnp.float32),
        scratch_types=[
            pltpu.VMEM((C,), jnp.int32),
            pltpu.VMEM((C,), jnp.int32),
            pltpu.VMEM((C,), jnp.float32),
            pltpu.VMEM((C,), jnp.float32),
            pltpu.VMEM((C, DIM), jnp.float32),
            pltpu.VMEM((C, DIM), jnp.float32),
            pltpu.VMEM((C, DIM), jnp.float32),
            pltpu.VMEM((C, DIM), jnp.float32),
            pltpu.SemaphoreType.DMA,
            pltpu.SemaphoreType.DMA,
            pltpu.SemaphoreType.DMA,
            pltpu.SemaphoreType.DMA,
        ],
    )
    def sc_call(ids_h, base_h, prior_h, gate_h, out_h,
                idx0, idx1, gte0, gte1, bb0, bb1, pb0, pb1,
                gsem0, gsem1, ssem0, ssem1):
        wid = lax.axis_index("s") * nc + lax.axis_index("c")
        wbase = wid * rows_per_w
        slots = ((idx0, gte0, bb0, pb0, gsem0, ssem0),
                 (idx1, gte1, bb1, pb1, gsem1, ssem1))

        def fire_gathers(c, slot):
            idxb, gteb, bb, pb, gsem, _ = slot
            row0 = wbase + c * C
            pltpu.sync_copy(ids_h.at[pl.ds(row0, C)], idxb)
            for j in range(NSUB):
                sl = pl.ds(j * SUB, SUB)
                pltpu.async_copy(base_h.at[idxb.at[sl]], bb.at[sl, :], gsem)
                pltpu.async_copy(prior_h.at[idxb.at[sl]], pb.at[sl, :], gsem)
                pltpu.async_copy(gate_h.at[idxb.at[sl]], gteb.at[sl], gsem)

        def wait_gathers(slot):
            idxb, gteb, bb, pb, gsem, _ = slot
            for j in range(NSUB):
                sl = pl.ds(j * SUB, SUB)
                pltpu.make_async_copy(
                    base_h.at[idxb.at[sl]], bb.at[sl, :], gsem).wait()
                pltpu.make_async_copy(
                    prior_h.at[idxb.at[sl]], pb.at[sl, :], gsem).wait()
                pltpu.make_async_copy(
                    gate_h.at[idxb.at[sl]], gteb.at[sl], gsem).wait()

        def compute(slot):
            _, gteb, bb, pb, _, _ = slot

            def group(i, carry):
                g16 = gteb[pl.ds(i * 16, 16)]
                w16 = G_MIN + (1.0 - G_MIN) / (1.0 + jnp.exp(-g16))
                for r in range(16):
                    row = i * 16 + r
                    wr = lax.gather(
                        w16, jnp.full((16, 1), r, jnp.int32), _DNUMS, (1,),
                        mode=lax.GatherScatterMode.PROMISE_IN_BOUNDS)
                    for dc in range(DIM // 16):
                        dsl = pl.ds(dc * 16, 16)
                        bb[row, dsl] = bb[row, dsl] + wr * pb[row, dsl]
                return carry

            lax.fori_loop(0, C // 16, group, 0)

        def fire_store(c, slot):
            _, _, bb, _, _, ssem = slot
            row0 = wbase + c * C
            pltpu.async_copy(bb, out_h.at[pl.ds(row0, C)], ssem)

        def wait_store(c, slot):
            _, _, bb, _, _, ssem = slot
            row0 = wbase + c * C
            pltpu.make_async_copy(bb, out_h.at[pl.ds(row0, C)], ssem).wait()

        fire_gathers(0, slots[0])
        fire_gathers(1, slots[1])

        def pair(p, carry):
            c0 = 2 * p
            c1 = c0 + 1
            wait_gathers(slots[0])
            compute(slots[0])
            fire_store(c0, slots[0])

            @pl.when(p < npairs - 1)
            def _():
                wait_store(c0, slots[0])
                fire_gathers(c0 + 2, slots[0])

            wait_gathers(slots[1])
            compute(slots[1])
            fire_store(c1, slots[1])

            @pl.when(p < npairs - 1)
            def _():
                wait_store(c1, slots[1])
                fire_gathers(c1 + 2, slots[1])

            return carry

        lax.fori_loop(0, npairs, pair, 0)
        wait_store(nchunks - 2, slots[0])
        wait_store(nchunks - 1, slots[1])

    return sc_call


_DETILE = _build_detile()
_GATHER = _build_gather()


@jax.jit
def kernel(input_ids, base_weight, prior_matrix, gate_logits):
    baseT = base_weight.T
    priorT = prior_matrix.T
    btail = base_weight[V_TAIL:, :].reshape(-1)
    ptail = prior_matrix[V_TAIL:, :].reshape(-1)
    base_flat, prior_flat = _DETILE(baseT, priorT, btail, ptail)
    base_rm = base_flat.reshape(VOCAB, DIM)
    prior_rm = prior_flat.reshape(VOCAB, DIM)
    ids_flat = input_ids.reshape(-1).astype(jnp.int32)
    out = _GATHER(ids_flat, base_rm, prior_rm, gate_logits)
    return out.reshape(*input_ids.shape, DIM)


# speedup vs baseline: 2.0879x; 2.0879x over previous
import functools

import jax
import jax.numpy as jnp
from jax import lax
from jax.experimental import pallas as pl
from jax.experimental.pallas import tpu as pltpu, tpu_sc as plsc

VOCAB = 1000000
DIM = 64
G_MIN = 0.1

N_ROWS = 4096 * 200

C = 256
SUB = 128
NSUB = C // SUB


def _mesh():
    return plsc.VectorSubcoreMesh(core_axis_name="c", subcore_axis_name="s")


def _build_gather():
    info = plsc.get_sparse_core_info()
    nc, ns = info.num_cores, info.num_subcores
    nw = nc * ns
    rows_per_w = N_ROWS // nw
    nchunks = rows_per_w // C
    npairs = nchunks // 2

    _DNUMS = lax.GatherDimensionNumbers(
        offset_dims=(), collapsed_slice_dims=(0,), start_index_map=(0,))

    @functools.partial(
        pl.kernel,
        mesh=_mesh(),
        compiler_params=pltpu.CompilerParams(use_tc_tiling_on_sc=False),
        out_type=jax.ShapeDtypeStruct((N_ROWS, DIM), jnp.float32),
        scratch_types=[
            pltpu.VMEM((C,), jnp.int32),
            pltpu.VMEM((C,), jnp.int32),
            pltpu.VMEM((C,), jnp.float32),
            pltpu.VMEM((C,), jnp.float32),
            pltpu.VMEM((C, DIM), jnp.float32),
            pltpu.VMEM((C, DIM), jnp.float32),
            pltpu.VMEM((C, DIM), jnp.float32),
            pltpu.VMEM((C, DIM), jnp.float32),
            pltpu.SemaphoreType.DMA,
            pltpu.SemaphoreType.DMA,
            pltpu.SemaphoreType.DMA,
            pltpu.SemaphoreType.DMA,
        ],
    )
    def sc_call(ids_h, base_h, prior_h, gate_h, out_h,
                idx0, idx1, gte0, gte1, bb0, bb1, pb0, pb1,
                gsem0, gsem1, ssem0, ssem1):
        wid = lax.axis_index("s") * nc + lax.axis_index("c")
        wbase = wid * rows_per_w
        slots = ((idx0, gte0, bb0, pb0, gsem0, ssem0),
                 (idx1, gte1, bb1, pb1, gsem1, ssem1))

        def fire_gathers(c, slot):
            idxb, gteb, bb, pb, gsem, _ = slot
            row0 = wbase + c * C
            pltpu.sync_copy(ids_h.at[pl.ds(row0, C)], idxb)
            for j in range(NSUB):
                sl = pl.ds(j * SUB, SUB)
                pltpu.async_copy(base_h.at[idxb.at[sl]], bb.at[sl, :], gsem)
                pltpu.async_copy(prior_h.at[idxb.at[sl]], pb.at[sl, :], gsem)
                pltpu.async_copy(gate_h.at[idxb.at[sl]], gteb.at[sl], gsem)

        def wait_gathers(slot):
            idxb, gteb, bb, pb, gsem, _ = slot
            for j in range(NSUB):
                sl = pl.ds(j * SUB, SUB)
                pltpu.make_async_copy(
                    base_h.at[idxb.at[sl]], bb.at[sl, :], gsem).wait()
                pltpu.make_async_copy(
                    prior_h.at[idxb.at[sl]], pb.at[sl, :], gsem).wait()
                pltpu.make_async_copy(
                    gate_h.at[idxb.at[sl]], gteb.at[sl], gsem).wait()

        def compute(slot):
            _, gteb, bb, pb, _, _ = slot

            def group(i, carry):
                g16 = gteb[pl.ds(i * 16, 16)]
                w16 = G_MIN + (1.0 - G_MIN) / (1.0 + jnp.exp(-g16))
                for r in range(16):
                    row = i * 16 + r
                    wr = lax.gather(
                        w16, jnp.full((16, 1), r, jnp.int32), _DNUMS, (1,),
                        mode=lax.GatherScatterMode.PROMISE_IN_BOUNDS)
                    for dc in range(DIM // 16):
                        dsl = pl.ds(dc * 16, 16)
                        bb[row, dsl] = bb[row, dsl] + wr * pb[row, dsl]
                return carry

            lax.fori_loop(0, C // 16, group, 0)

        def fire_store(c, slot):
            _, _, bb, _, _, ssem = slot
            row0 = wbase + c * C
            pltpu.async_copy(bb, out_h.at[pl.ds(row0, C)], ssem)

        def wait_store(c, slot):
            _, _, bb, _, _, ssem = slot
            row0 = wbase + c * C
            pltpu.make_async_copy(bb, out_h.at[pl.ds(row0, C)], ssem).wait()

        fire_gathers(0, slots[0])
        fire_gathers(1, slots[1])

        def pair(p, carry):
            c0 = 2 * p
            c1 = c0 + 1
            wait_gathers(slots[0])
            compute(slots[0])
            fire_store(c0, slots[0])

            @pl.when(p < npairs - 1)
            def _():
                wait_store(c0, slots[0])
                fire_gathers(c0 + 2, slots[0])

            wait_gathers(slots[1])
            compute(slots[1])
            fire_store(c1, slots[1])

            @pl.when(p < npairs - 1)
            def _():
                wait_store(c1, slots[1])
                fire_gathers(c1 + 2, slots[1])

            return carry

        lax.fori_loop(0, npairs, pair, 0)
        wait_store(nchunks - 2, slots[0])
        wait_store(nchunks - 1, slots[1])

    return sc_call


_GATHER = _build_gather()


@jax.jit
def kernel(input_ids, base_weight, prior_matrix, gate_logits):
    ids_flat = input_ids.reshape(-1).astype(jnp.int32)
    out = _GATHER(ids_flat, base_weight, prior_matrix, gate_logits)
    return out.reshape(*input_ids.shape, DIM)
